# baseline (speedup 1.0000x reference)
import functools

import jax
import jax.numpy as jnp
from jax import lax
from jax.experimental import pallas as pl
from jax.experimental.pallas import tpu as pltpu
from jax.experimental.pallas import tpu_sc as plsc

_NBUF = 4
_LANES = 16
_BBLK = 128


@functools.lru_cache(maxsize=None)
def _build(batch, seq_len, vocab, d):
    info = plsc.get_sparse_core_info()
    nc, ns = info.num_cores, info.num_subcores
    nw = nc * ns
    assert batch == nw * _BBLK
    assert seq_len % _NBUF == 0
    assert d % 8 == 0
    dt8 = d // 8

    mesh = plsc.VectorSubcoreMesh(core_axis_name="c", subcore_axis_name="s")

    @functools.partial(
        pl.kernel,
        mesh=mesh,
        compiler_params=pltpu.CompilerParams(use_tc_tiling_on_sc=False,
                                             needs_layout_passes=False),
        out_type=jax.ShapeDtypeStruct((seq_len, dt8, nw, 8, _BBLK), jnp.float32),
        scratch_types=[
            pltpu.VMEM((_NBUF, _BBLK), jnp.int32),
            pltpu.VMEM((_NBUF, _BBLK, d), jnp.float32),
            pltpu.VMEM((_NBUF, dt8, 8, _BBLK + 1), jnp.float32),
            pltpu.VMEM((seq_len, d), jnp.float32),
            [pltpu.SemaphoreType.DMA] * _NBUF,
            [pltpu.SemaphoreType.DMA] * _NBUF,
            [pltpu.SemaphoreType.DMA] * _NBUF,
        ],
    )
    def emb(xt_hbm, tok_hbm, pos_hbm, out_hbm, idx_v, rows_v, tbuf_v, pos_v,
            gsems, ssems, isems):
        w = lax.axis_index("s") * nc + lax.axis_index("c")
        col0 = w * _BBLK
        pltpu.sync_copy(pos_hbm, pos_v)

        def fire_idx(p, b):
            pltpu.async_copy(xt_hbm.at[p, pl.ds(col0, _BBLK)], idx_v.at[b],
                             isems[b])

        def wait_idx(b):
            pltpu.make_async_copy(xt_hbm.at[0, pl.ds(0, _BBLK)], idx_v.at[b],
                                  isems[b]).wait()

        def fire_gather(b):
            pltpu.async_copy(tok_hbm.at[idx_v.at[b]], rows_v.at[b], gsems[b])

        def wait_gather(b):
            pltpu.make_async_copy(tok_hbm.at[idx_v.at[b]], rows_v.at[b],
                                  gsems[b]).wait()

        def fire_store(p, b):
            pltpu.async_copy(tbuf_v.at[b, :, :, pl.ds(0, _BBLK)],
                             out_hbm.at[p, :, w, :, :], ssems[b])

        def wait_store(b):
            pltpu.make_async_copy(tbuf_v.at[b, :, :, pl.ds(0, _BBLK)],
                                  out_hbm.at[0, :, 0, :, :], ssems[b]).wait()

        iota16 = lax.iota(jnp.int32, _LANES)
        dt_idxs = [iota16 // 8 + k * 2 for k in range(d // _LANES)]
        ds_idx = iota16 % 8

        fire_idx(0, 0)
        fire_idx(1, 1)
        wait_idx(0)
        fire_gather(0)

        def super_body(t, carry):
            for b in range(_NBUF):
                p = t * _NBUF + b
                b1 = (b + 1) % _NBUF
                b2 = (b + 2) % _NBUF

                @pl.when(p + 1 < seq_len)
                def _():
                    wait_idx(b1)
                    fire_gather(b1)

                @pl.when(p + 2 < seq_len)
                def _():
                    fire_idx(p + 2, b2)

                wait_gather(b)

                @pl.when(p >= _NBUF)
                def _():
                    wait_store(b)

                pos_ks = [pos_v[p, pl.ds(k * _LANES, _LANES)]
                          for k in range(d // _LANES)]

                def row_body(r, carry2):
                    rvec = jnp.full((_LANES,), r, dtype=jnp.int32)
                    for k in range(d // _LANES):
                        vec = rows_v[b, r, pl.ds(k * _LANES, _LANES)]
                        plsc.store_scatter(tbuf_v.at[b],
                                           [dt_idxs[k], ds_idx, rvec],
                                           vec + pos_ks[k])
                    return carry2

                lax.fori_loop(0, _BBLK, row_body, 0)
                fire_store(p, b)
            return carry

        lax.fori_loop(0, seq_len // _NBUF, super_body, 0)
        for b in range(_NBUF):
            wait_store(b)

    return emb


def kernel(x, token_table, pos_table):
    batch, seq_len = x.shape
    vocab, d = token_table.shape
    xt = jnp.transpose(x.astype(jnp.int32))
    emb = _build(batch, seq_len, vocab, d)
    out5 = emb(xt, token_table.astype(jnp.float32),
               pos_table.astype(jnp.float32))
    return jnp.transpose(out5, (2, 4, 0, 1, 3)).reshape(batch, seq_len, d)


# speedup vs baseline: 2.2929x; 2.2929x over previous
import functools

import jax
import jax.numpy as jnp
from jax import lax
from jax.experimental import pallas as pl
from jax.experimental.pallas import tpu as pltpu
from jax.experimental.pallas import tpu_sc as plsc

_NBUF = 4
_LANES = 16
_BBLK = 128


@functools.lru_cache(maxsize=None)
def _build(batch, seq_len, vocab, d):
    info = plsc.get_sparse_core_info()
    nc, ns = info.num_cores, info.num_subcores
    nw = nc * ns
    assert batch == nw * _BBLK
    assert seq_len % _NBUF == 0
    assert d % 8 == 0
    dt8 = d // 8

    mesh = plsc.VectorSubcoreMesh(core_axis_name="c", subcore_axis_name="s")

    @functools.partial(
        pl.kernel,
        mesh=mesh,
        compiler_params=pltpu.CompilerParams(use_tc_tiling_on_sc=False,
                                             needs_layout_passes=False),
        out_type=jax.ShapeDtypeStruct((seq_len, dt8, nw, 8, _BBLK), jnp.float32),
        scratch_types=[
            pltpu.VMEM((_NBUF, _BBLK), jnp.int32),
            pltpu.VMEM((_NBUF, _BBLK, d), jnp.float32),
            pltpu.VMEM((_NBUF, dt8, 8, _BBLK + 1), jnp.float32),
            pltpu.VMEM((seq_len, d), jnp.float32),
            [pltpu.SemaphoreType.DMA] * _NBUF,
            [pltpu.SemaphoreType.DMA] * _NBUF,
            [pltpu.SemaphoreType.DMA] * _NBUF,
        ],
    )
    def emb(xt_hbm, tok_hbm, pos_hbm, out_hbm, idx_v, rows_v, tbuf_v, pos_v,
            gsems, ssems, isems):
        w = lax.axis_index("s") * nc + lax.axis_index("c")
        col0 = w * _BBLK
        pltpu.sync_copy(pos_hbm, pos_v)

        def fire_idx(p, b):
            pltpu.async_copy(xt_hbm.at[p, pl.ds(col0, _BBLK)], idx_v.at[b],
                             isems[b])

        def wait_idx(b):
            pltpu.make_async_copy(xt_hbm.at[0, pl.ds(0, _BBLK)], idx_v.at[b],
                                  isems[b]).wait()

        def fire_gather(b):
            pltpu.async_copy(tok_hbm.at[idx_v.at[b]], rows_v.at[b], gsems[b])

        def wait_gather(b):
            pltpu.make_async_copy(tok_hbm.at[idx_v.at[b]], rows_v.at[b],
                                  gsems[b]).wait()

        def fire_store(p, b):
            pltpu.async_copy(tbuf_v.at[b, :, :, pl.ds(0, _BBLK)],
                             out_hbm.at[p, :, w, :, :], ssems[b])

        def wait_store(b):
            pltpu.make_async_copy(tbuf_v.at[b, :, :, pl.ds(0, _BBLK)],
                                  out_hbm.at[0, :, 0, :, :], ssems[b]).wait()

        iota16 = lax.iota(jnp.int32, _LANES)
        dt_idxs = [iota16 // 8 + k * 2 for k in range(d // _LANES)]
        ds_idx = iota16 % 8

        fire_idx(0, 0)
        fire_idx(1, 1)
        wait_idx(0)
        fire_gather(0)

        def super_body(t, carry):
            for b in range(_NBUF):
                p = t * _NBUF + b
                b1 = (b + 1) % _NBUF
                b2 = (b + 2) % _NBUF

                @pl.when(p + 1 < seq_len)
                def _():
                    wait_idx(b1)
                    fire_gather(b1)

                @pl.when(p + 2 < seq_len)
                def _():
                    fire_idx(p + 2, b2)

                wait_gather(b)

                @pl.when(p >= _NBUF)
                def _():
                    wait_store(b)

                pos_ks = [pos_v[p, pl.ds(k * _LANES, _LANES)]
                          for k in range(d // _LANES)]

                @plsc.parallel_loop(0, _BBLK, step=1, unroll=4)
                def row_body(r):
                    rvec = jnp.full((_LANES,), r, dtype=jnp.int32)
                    for k in range(d // _LANES):
                        vec = rows_v[b, r, pl.ds(k * _LANES, _LANES)]
                        plsc.store_scatter(tbuf_v.at[b],
                                           [dt_idxs[k], ds_idx, rvec],
                                           vec + pos_ks[k])
                fire_store(p, b)
            return carry

        lax.fori_loop(0, seq_len // _NBUF, super_body, 0)
        for b in range(_NBUF):
            wait_store(b)

    return emb


def kernel(x, token_table, pos_table):
    batch, seq_len = x.shape
    vocab, d = token_table.shape
    xt = jnp.transpose(x.astype(jnp.int32))
    emb = _build(batch, seq_len, vocab, d)
    out5 = emb(xt, token_table.astype(jnp.float32),
               pos_table.astype(jnp.float32))
    return jnp.transpose(out5, (2, 4, 0, 1, 3)).reshape(batch, seq_len, d)


# speedup vs baseline: 2.5315x; 1.1041x over previous
import functools

import jax
import jax.numpy as jnp
from jax import lax
from jax.experimental import pallas as pl
from jax.experimental.pallas import tpu as pltpu
from jax.experimental.pallas import tpu_sc as plsc

_NBUF = 4
_LANES = 16
_BBLK = 128


@functools.lru_cache(maxsize=None)
def _build(batch, seq_len, vocab, d):
    info = plsc.get_sparse_core_info()
    nc, ns = info.num_cores, info.num_subcores
    nw = nc * ns
    assert batch == nw * _BBLK
    assert seq_len % 8 == 0 and seq_len % _NBUF == 0
    assert d % _LANES == 0
    dt8 = d // 8

    mesh = plsc.VectorSubcoreMesh(core_axis_name="c", subcore_axis_name="s")

    @functools.partial(
        pl.kernel,
        mesh=mesh,
        compiler_params=pltpu.CompilerParams(use_tc_tiling_on_sc=False,
                                             needs_layout_passes=False),
        out_type=jax.ShapeDtypeStruct((seq_len, dt8, nw, 8, _BBLK), jnp.float32),
        scratch_types=[
            pltpu.VMEM((_NBUF, _BBLK), jnp.int32),
            pltpu.VMEM((_NBUF, _BBLK, d), jnp.float32),
            pltpu.VMEM((_NBUF, dt8, 8, _BBLK + 1), jnp.float32),
            pltpu.VMEM((seq_len, d), jnp.float32),
            [pltpu.SemaphoreType.DMA] * _NBUF,
            [pltpu.SemaphoreType.DMA] * _NBUF,
            [pltpu.SemaphoreType.DMA] * _NBUF,
        ],
    )
    def emb(x4_hbm, tok_hbm, pos_hbm, out_hbm, idx_v, rows_v, tbuf_v, pos_v,
            gsems, ssems, isems):
        w = lax.axis_index("s") * nc + lax.axis_index("c")
        pltpu.sync_copy(pos_hbm, pos_v)

        def fire_idx(p, b):
            pltpu.async_copy(x4_hbm.at[p // 8, w, p % 8, :], idx_v.at[b],
                             isems[b])

        def wait_idx(b):
            pltpu.make_async_copy(x4_hbm.at[0, 0, 0, :], idx_v.at[b],
                                  isems[b]).wait()

        def fire_gather(b):
            pltpu.async_copy(tok_hbm.at[idx_v.at[b]], rows_v.at[b], gsems[b])

        def wait_gather(b):
            pltpu.make_async_copy(tok_hbm.at[idx_v.at[b]], rows_v.at[b],
                                  gsems[b]).wait()

        def fire_store(p, b):
            pltpu.async_copy(tbuf_v.at[b, :, :, pl.ds(0, _BBLK)],
                             out_hbm.at[p, :, w, :, :], ssems[b])

        def wait_store(b):
            pltpu.make_async_copy(tbuf_v.at[b, :, :, pl.ds(0, _BBLK)],
                                  out_hbm.at[0, :, 0, :, :], ssems[b]).wait()

        iota16 = lax.iota(jnp.int32, _LANES)
        dt_idxs = [iota16 // 8 + k * 2 for k in range(d // _LANES)]
        ds_idx = iota16 % 8

        fire_idx(0, 0)
        fire_idx(1, 1)
        fire_idx(2, 2)
        wait_idx(0)
        fire_gather(0)
        wait_idx(1)
        fire_gather(1)

        def super_body(t, carry):
            for b in range(_NBUF):
                p = t * _NBUF + b
                b2 = (b + 2) % _NBUF
                b3 = (b + 3) % _NBUF

                wait_gather(b)

                @pl.when(p + 2 < seq_len)
                def _():
                    wait_idx(b2)
                    fire_gather(b2)

                @pl.when(p + 3 < seq_len)
                def _():
                    fire_idx(p + 3, b3)

                @pl.when(p >= _NBUF)
                def _():
                    wait_store(b)

                pos_ks = [pos_v[p, pl.ds(k * _LANES, _LANES)]
                          for k in range(d // _LANES)]

                @plsc.parallel_loop(0, _BBLK, step=1, unroll=4)
                def row_body(r):
                    rvec = jnp.full((_LANES,), r, dtype=jnp.int32)
                    for k in range(d // _LANES):
                        vec = rows_v[b, r, pl.ds(k * _LANES, _LANES)]
                        plsc.store_scatter(tbuf_v.at[b],
                                           [dt_idxs[k], ds_idx, rvec],
                                           vec + pos_ks[k])

                fire_store(p, b)
            return carry

        lax.fori_loop(0, seq_len // _NBUF, super_body, 0)
        for b in range(_NBUF):
            wait_store(b)

    return emb


def kernel(x, token_table, pos_table):
    batch, seq_len = x.shape
    vocab, d = token_table.shape
    x4 = jnp.transpose(
        jnp.transpose(x.astype(jnp.int32)).reshape(seq_len // 8, 8,
                                                   batch // 128, 128),
        (0, 2, 1, 3))
    emb = _build(batch, seq_len, vocab, d)
    out5 = emb(x4, token_table.astype(jnp.float32),
               pos_table.astype(jnp.float32))
    return jnp.transpose(out5, (2, 4, 0, 1, 3)).reshape(batch, seq_len, d)


# speedup vs baseline: 2.6557x; 1.0491x over previous
import functools

import jax
import jax.numpy as jnp
from jax import lax
from jax.experimental import pallas as pl
from jax.experimental.pallas import tpu as pltpu
from jax.experimental.pallas import tpu_sc as plsc

_NBUF = 4
_LANES = 16
_BBLK = 128


@functools.lru_cache(maxsize=None)
def _build(batch, seq_len, vocab, d):
    info = plsc.get_sparse_core_info()
    nc, ns = info.num_cores, info.num_subcores
    nw = nc * ns
    assert batch == nw * _BBLK
    assert seq_len % 8 == 0 and seq_len % _NBUF == 0
    assert d % _LANES == 0
    dt8 = d // 8

    mesh = plsc.VectorSubcoreMesh(core_axis_name="c", subcore_axis_name="s")

    @functools.partial(
        pl.kernel,
        mesh=mesh,
        compiler_params=pltpu.CompilerParams(use_tc_tiling_on_sc=False,
                                             needs_layout_passes=False),
        out_type=jax.ShapeDtypeStruct((seq_len, dt8, nw, 8, _BBLK), jnp.float32),
        scratch_types=[
            pltpu.VMEM((_NBUF, _BBLK), jnp.int32),
            pltpu.VMEM((_NBUF, _BBLK, d), jnp.float32),
            pltpu.VMEM((_NBUF, dt8, 8, _BBLK + 1), jnp.float32),
            pltpu.VMEM((seq_len, d), jnp.float32),
            [pltpu.SemaphoreType.DMA] * _NBUF,
            [pltpu.SemaphoreType.DMA] * _NBUF,
            [pltpu.SemaphoreType.DMA] * _NBUF,
        ],
    )
    def emb(x4_hbm, tok_hbm, pos_hbm, out_hbm, idx_v, rows_v, tbuf_v, pos_v,
            gsems, ssems, isems):
        w = lax.axis_index("s") * nc + lax.axis_index("c")
        pltpu.sync_copy(pos_hbm, pos_v)

        def fire_idx(p, b):
            pltpu.async_copy(x4_hbm.at[p // 8, w, p % 8, :], idx_v.at[b],
                             isems[b])

        def wait_idx(b):
            pltpu.make_async_copy(x4_hbm.at[0, 0, 0, :], idx_v.at[b],
                                  isems[b]).wait()

        def fire_gather(b):
            pltpu.async_copy(tok_hbm.at[idx_v.at[b]], rows_v.at[b], gsems[b])

        def wait_gather(b):
            pltpu.make_async_copy(tok_hbm.at[idx_v.at[b]], rows_v.at[b],
                                  gsems[b]).wait()

        def fire_store(p, b):
            pltpu.async_copy(tbuf_v.at[b, :, :, pl.ds(0, _BBLK)],
                             out_hbm.at[p, :, w, :, :], ssems[b])

        def wait_store(b):
            pltpu.make_async_copy(tbuf_v.at[b, :, :, pl.ds(0, _BBLK)],
                                  out_hbm.at[0, :, 0, :, :], ssems[b]).wait()

        iota16 = lax.iota(jnp.int32, _LANES)
        dt_idxs = [iota16 // 8 + k * 2 for k in range(d // _LANES)]
        ds_idx = iota16 % 8

        fire_idx(0, 0)
        fire_idx(1, 1)
        fire_idx(2, 2)
        fire_idx(3, 3)
        wait_idx(0)
        fire_gather(0)
        wait_idx(1)
        fire_gather(1)
        wait_idx(2)
        fire_gather(2)

        def super_body(t, carry):
            for b in range(_NBUF):
                p = t * _NBUF + b
                b3 = (b + 3) % _NBUF

                wait_gather(b)

                @pl.when(p + 4 < seq_len)
                def _():
                    fire_idx(p + 4, b)

                @pl.when(p + 3 < seq_len)
                def _():
                    wait_idx(b3)
                    fire_gather(b3)

                @pl.when(p >= _NBUF)
                def _():
                    wait_store(b)

                pos_ks = [pos_v[p, pl.ds(k * _LANES, _LANES)]
                          for k in range(d // _LANES)]

                @plsc.parallel_loop(0, _BBLK, step=1, unroll=4)
                def row_body(r):
                    rvec = jnp.full((_LANES,), r, dtype=jnp.int32)
                    for k in range(d // _LANES):
                        vec = rows_v[b, r, pl.ds(k * _LANES, _LANES)]
                        plsc.store_scatter(tbuf_v.at[b],
                                           [dt_idxs[k], ds_idx, rvec],
                                           vec + pos_ks[k])

                fire_store(p, b)
            return carry

        lax.fori_loop(0, seq_len // _NBUF, super_body, 0)
        for b in range(_NBUF):
            wait_store(b)

    return emb


def kernel(x, token_table, pos_table):
    batch, seq_len = x.shape
    vocab, d = token_table.shape
    x4 = jnp.transpose(
        jnp.transpose(x.astype(jnp.int32)).reshape(seq_len // 8, 8,
                                                   batch // 128, 128),
        (0, 2, 1, 3))
    emb = _build(batch, seq_len, vocab, d)
    out5 = emb(x4, token_table.astype(jnp.float32),
               pos_table.astype(jnp.float32))
    return jnp.transpose(out5, (2, 4, 0, 1, 3)).reshape(batch, seq_len, d)


# speedup vs baseline: 2.7537x; 1.0369x over previous
import functools

import jax
import jax.numpy as jnp
from jax import lax
from jax.experimental import pallas as pl
from jax.experimental.pallas import tpu as pltpu
from jax.experimental.pallas import tpu_sc as plsc

_NBUF = 4
_LANES = 16
_BBLK = 128


@functools.lru_cache(maxsize=None)
def _build(batch, seq_len, vocab, d):
    info = plsc.get_sparse_core_info()
    nc, ns = info.num_cores, info.num_subcores
    nw = nc * ns
    assert batch == nw * _BBLK
    assert seq_len % 8 == 0 and seq_len % _NBUF == 0
    assert d % _LANES == 0 and 128 % d == 0
    dt8 = d // 8

    mesh = plsc.VectorSubcoreMesh(core_axis_name="c", subcore_axis_name="s")

    @functools.partial(
        pl.kernel,
        mesh=mesh,
        compiler_params=pltpu.CompilerParams(use_tc_tiling_on_sc=False,
                                             needs_layout_passes=False),
        out_type=jax.ShapeDtypeStruct((seq_len, dt8, nw, 8, _BBLK), jnp.float32),
        scratch_types=[
            pltpu.VMEM((_NBUF, _BBLK), jnp.int32),
            pltpu.VMEM((_NBUF, _BBLK, d), jnp.float32),
            pltpu.VMEM((_NBUF, dt8, 8, _BBLK + 1), jnp.float32),
            pltpu.VMEM((seq_len, d), jnp.float32),
            [pltpu.SemaphoreType.DMA] * _NBUF,
            [pltpu.SemaphoreType.DMA] * _NBUF,
            [pltpu.SemaphoreType.DMA] * _NBUF,
        ],
    )
    def emb(x4_hbm, tok_hbm, pos_hbm, out_hbm, idx_v, rows_v, tbuf_v, pos_v,
            gsems, ssems, isems):
        w = lax.axis_index("s") * nc + lax.axis_index("c")
        pltpu.sync_copy(pos_hbm, pos_v)

        def fire_idx(p, b):
            pltpu.async_copy(x4_hbm.at[p // 8, w, p % 8, :], idx_v.at[b],
                             isems[b])

        def wait_idx(b):
            pltpu.make_async_copy(x4_hbm.at[0, 0, 0, :], idx_v.at[b],
                                  isems[b]).wait()

        def fire_gather(b):
            for k in range(_BBLK // _LANES):
                sl = pl.ds(k * _LANES, _LANES)
                idx_v[b, sl] = idx_v[b, sl] + idx_v[b, sl]
            pltpu.async_copy(tok_hbm.at[idx_v.at[b]], rows_v.at[b], gsems[b])

        def wait_gather(b):
            pltpu.make_async_copy(tok_hbm.at[idx_v.at[b]], rows_v.at[b],
                                  gsems[b]).wait()

        def fire_store(p, b):
            pltpu.async_copy(tbuf_v.at[b, :, :, pl.ds(0, _BBLK)],
                             out_hbm.at[p, :, w, :, :], ssems[b])

        def wait_store(b):
            pltpu.make_async_copy(tbuf_v.at[b, :, :, pl.ds(0, _BBLK)],
                                  out_hbm.at[0, :, 0, :, :], ssems[b]).wait()

        iota16 = lax.iota(jnp.int32, _LANES)
        dt_idxs = [iota16 // 8 + k * 2 for k in range(d // _LANES)]
        ds_idx = iota16 % 8

        fire_idx(0, 0)
        fire_idx(1, 1)
        fire_idx(2, 2)
        fire_idx(3, 3)
        wait_idx(0)
        fire_gather(0)
        wait_idx(1)
        fire_gather(1)
        wait_idx(2)
        fire_gather(2)

        def super_body(t, carry):
            for b in range(_NBUF):
                p = t * _NBUF + b
                b3 = (b + 3) % _NBUF

                wait_gather(b)

                @pl.when(p + 4 < seq_len)
                def _():
                    fire_idx(p + 4, b)

                @pl.when(p + 3 < seq_len)
                def _():
                    wait_idx(b3)
                    fire_gather(b3)

                @pl.when(p >= _NBUF)
                def _():
                    wait_store(b)

                pos_ks = [pos_v[p, pl.ds(k * _LANES, _LANES)]
                          for k in range(d // _LANES)]

                @plsc.parallel_loop(0, _BBLK, step=1, unroll=4)
                def row_body(r):
                    rvec = jnp.full((_LANES,), r, dtype=jnp.int32)
                    for k in range(d // _LANES):
                        vec = rows_v[b, r, pl.ds(k * _LANES, _LANES)]
                        plsc.store_scatter(tbuf_v.at[b],
                                           [dt_idxs[k], ds_idx, rvec],
                                           vec + pos_ks[k])

                fire_store(p, b)
            return carry

        lax.fori_loop(0, seq_len // _NBUF, super_body, 0)
        for b in range(_NBUF):
            wait_store(b)

    return emb


def kernel(x, token_table, pos_table):
    batch, seq_len = x.shape
    vocab, d = token_table.shape
    x4 = jnp.transpose(
        jnp.transpose(x.astype(jnp.int32)).reshape(seq_len // 8, 8,
                                                   batch // 128, 128),
        (0, 2, 1, 3))
    tabp = jnp.pad(token_table.astype(jnp.float32),
                   ((0, 0), (0, 128 - d))).reshape(2 * vocab, d)
    emb = _build(batch, seq_len, vocab, d)
    out5 = emb(x4, tabp, pos_table.astype(jnp.float32))
    return jnp.transpose(out5, (2, 4, 0, 1, 3)).reshape(batch, seq_len, d)
